# Initial kernel scaffold; baseline (speedup 1.0000x reference)
import jax
import jax.numpy as jnp
from jax.experimental import pallas as pl


def kernel(x, edge_index, batch_size, gin_W1, gin_b1, gin_bn_g, gin_bn_b, gin_W2, gin_b2, norm_g, norm_b, jk_W, jk_b, lin_W, lin_b, v_W, v_b, pi_W, pi_b):
    raise NotImplementedError("write your pallas kernel here")


# speedup vs baseline: 5.1735x; 5.1735x over previous
import functools

import jax
import jax.numpy as jnp
from jax import lax
from jax.experimental import pallas as pl
from jax.experimental.pallas import tpu as pltpu
from jax.experimental.pallas import tpu_sc as plsc

N = 10000
E = 320000
D = 128
DEPTH = 3
BATCH = 10
GPN = N // BATCH

NC = 2
NS = 16
NW = NC * NS
EPW = E // NW
CH = 128
NFULL = EPW // CH
TAIL = EPW - NFULL * CH
NP = 10240
RPT = NP // NS
ZR = 128


def _seg_partials(h, src, dst):
  mesh = plsc.VectorSubcoreMesh(core_axis_name="c", subcore_axis_name="s")

  @functools.partial(
      pl.kernel,
      mesh=mesh,
      out_type=jax.ShapeDtypeStruct((NC * NP, D), jnp.float32),
      scratch_types=[
          pltpu.VMEM((CH,), jnp.int32),
          pltpu.VMEM((CH,), jnp.int32),
          pltpu.VMEM((CH, D), jnp.float32),
          pltpu.VMEM((TAIL,), jnp.int32),
          pltpu.VMEM((TAIL,), jnp.int32),
          pltpu.VMEM((TAIL, D), jnp.float32),
          pltpu.VMEM((ZR, D), jnp.float32),
          pltpu.VMEM_SHARED((NP, D), jnp.float32),
          pltpu.SemaphoreType.DMA,
      ],
  )
  def k(h_hbm, src_hbm, dst_hbm, out_hbm,
        sidx, didx, rows, sidx_t, didx_t, rows_t, zbuf, acc, sem):
    cid = lax.axis_index("c")
    sid = lax.axis_index("s")
    wid = sid * NC + cid

    zeros16 = jnp.zeros((16,), jnp.float32)

    def zb(i, carry):
      r = i // (D // 16)
      c = i % (D // 16)
      zbuf[r, pl.ds(c * 16, 16)] = zeros16
      return carry

    lax.fori_loop(0, ZR * (D // 16), zb, 0)

    def zc(i, carry):
      pltpu.sync_copy(zbuf, acc.at[pl.ds(sid * RPT + i * ZR, ZR)])
      return carry

    lax.fori_loop(0, RPT // ZR, zc, 0)
    plsc.subcore_barrier()

    ebase = wid * EPW

    def body(j, carry):
      off = ebase + j * CH
      pltpu.sync_copy(src_hbm.at[pl.ds(off, CH)], sidx)
      pltpu.sync_copy(dst_hbm.at[pl.ds(off, CH)], didx)
      pltpu.async_copy(h_hbm.at[sidx], rows, sem).wait()
      pltpu.sync_copy(rows, acc.at[didx], add=True)
      return carry

    lax.fori_loop(0, NFULL, body, 0)

    toff = ebase + NFULL * CH
    pltpu.sync_copy(src_hbm.at[pl.ds(toff, TAIL)], sidx_t)
    pltpu.sync_copy(dst_hbm.at[pl.ds(toff, TAIL)], didx_t)
    pltpu.async_copy(h_hbm.at[sidx_t], rows_t, sem).wait()
    pltpu.sync_copy(rows_t, acc.at[didx_t], add=True)

    plsc.subcore_barrier()

    def wo(i, carry):
      r0 = sid * RPT + i * ZR
      pltpu.sync_copy(acc.at[pl.ds(r0, ZR)], zbuf)
      pltpu.sync_copy(zbuf, out_hbm.at[pl.ds(cid * NP + r0, ZR)])
      return carry

    lax.fori_loop(0, RPT // ZR, wo, 0)

  return k(h, src, dst)


def _layer_body(h_ref, parts_ref, w1_ref, b1_ref, g1_ref, bb1_ref,
                w2_ref, b2_ref, g2_ref, bb2_ref, o_ref):
  z = h_ref[...] + parts_ref[0] + parts_ref[1]
  z = jnp.dot(z, w1_ref[...], preferred_element_type=jnp.float32) + b1_ref[...]
  mu = jnp.mean(z, axis=0, keepdims=True)
  var = jnp.mean(jnp.square(z - mu), axis=0, keepdims=True)
  z = (z - mu) * lax.rsqrt(var + 1e-5) * g1_ref[...] + bb1_ref[...]
  z = jnp.maximum(z, 0.0)
  z = jnp.dot(z, w2_ref[...], preferred_element_type=jnp.float32) + b2_ref[...]
  mu = jnp.mean(z, axis=0, keepdims=True)
  var = jnp.mean(jnp.square(z - mu), axis=0, keepdims=True)
  z = (z - mu) * lax.rsqrt(var + 1e-5) * g2_ref[...] + bb2_ref[...]
  o_ref[...] = jnp.maximum(z, 0.0)


def _layer_tc(h, parts, w1, b1, g1, bb1, w2, b2, g2, bb2):
  r = lambda a: a.reshape(1, D)
  wsp = lambda s: pl.BlockSpec(s, lambda i: (0,) * len(s))
  return pl.pallas_call(
      _layer_body,
      grid=(1,),
      in_specs=[
          wsp((N, D)),
          wsp((2, N, D)),
          wsp((D, D)), wsp((1, D)), wsp((1, D)), wsp((1, D)),
          wsp((D, D)), wsp((1, D)), wsp((1, D)), wsp((1, D)),
      ],
      out_specs=wsp((N, D)),
      out_shape=jax.ShapeDtypeStruct((N, D), jnp.float32),
  )(h, parts.reshape(2, NP, D), w1, r(b1), r(g1), r(bb1),
    w2, r(b2), r(g2), r(bb2))


def _head_body(x_ref, x1_ref, x2_ref, x3_ref, jkw_ref, jkb_ref,
               linw_ref, linb_ref, vw_ref, vb_ref, piw_ref, pib_ref,
               pi_ref, v_ref):
  xb = x_ref[0]
  hcat = jnp.concatenate([x1_ref[0], x2_ref[0], x3_ref[0]], axis=1)
  hcat = jnp.dot(hcat, jkw_ref[...],
                 preferred_element_type=jnp.float32) + jkb_ref[...]
  xfull = jnp.concatenate([xb, hcat], axis=1)
  feat = jnp.dot(xfull, linw_ref[...],
                 preferred_element_type=jnp.float32) + linb_ref[...]
  piv = jnp.sum(feat * piw_ref[...], axis=1) + pib_ref[0, 0]
  m = jnp.max(piv)
  lse = jnp.log(jnp.sum(jnp.exp(piv - m))) + m
  pi_ref[0, 0, :] = piv - lse
  fm = jnp.mean(feat, axis=0, keepdims=True)
  v = jnp.dot(fm, vw_ref[...], preferred_element_type=jnp.float32) + vb_ref[...]
  vm = jnp.max(v)
  vlse = jnp.log(jnp.sum(jnp.exp(v - vm))) + vm
  v_ref[0] = v - vlse


def _head_tc(x, x1, x2, x3, jk_w, jk_b, lin_w, lin_b, v_w, v_b, pi_w, pi_b):
  g3 = lambda g: (g, 0, 0)
  w0 = lambda g: (0, 0)
  DD = DEPTH * D
  return pl.pallas_call(
      _head_body,
      grid=(BATCH,),
      in_specs=[
          pl.BlockSpec((1, GPN, D), g3),
          pl.BlockSpec((1, GPN, D), g3),
          pl.BlockSpec((1, GPN, D), g3),
          pl.BlockSpec((1, GPN, D), g3),
          pl.BlockSpec((DD, DD), w0),
          pl.BlockSpec((1, DD), w0),
          pl.BlockSpec((D + DD, 32), w0),
          pl.BlockSpec((1, 32), w0),
          pl.BlockSpec((32, 3), w0),
          pl.BlockSpec((1, 3), w0),
          pl.BlockSpec((1, 32), w0),
          pl.BlockSpec((1, 1), w0),
      ],
      out_specs=[
          pl.BlockSpec((1, 1, GPN), g3),
          pl.BlockSpec((1, 1, 3), g3),
      ],
      out_shape=[
          jax.ShapeDtypeStruct((BATCH, 1, GPN), jnp.float32),
          jax.ShapeDtypeStruct((BATCH, 1, 3), jnp.float32),
      ],
  )(x.reshape(BATCH, GPN, D), x1.reshape(BATCH, GPN, D),
    x2.reshape(BATCH, GPN, D), x3.reshape(BATCH, GPN, D),
    jk_w, jk_b.reshape(1, DD), lin_w, lin_b.reshape(1, 32),
    v_w, v_b.reshape(1, 3), pi_w.reshape(1, 32), pi_b.reshape(1, 1))


def kernel(x, edge_index, batch_size, gin_W1, gin_b1, gin_bn_g, gin_bn_b,
           gin_W2, gin_b2, norm_g, norm_b, jk_W, jk_b, lin_W, lin_b,
           v_W, v_b, pi_W, pi_b):
  src = edge_index[0]
  dst = edge_index[1]
  h = x
  xs = []
  for i in range(DEPTH):
    parts = _seg_partials(h, src, dst)
    h = _layer_tc(h, parts, gin_W1[i], gin_b1[i],
                  gin_bn_g[i], gin_bn_b[i], gin_W2[i], gin_b2[i],
                  norm_g[i], norm_b[i])
    xs.append(h)
  pi, v = _head_tc(x, xs[0], xs[1], xs[2], jk_W, jk_b, lin_W, lin_b,
                   v_W, v_b, pi_W, pi_b)
  return (pi.reshape(BATCH, GPN), v.reshape(BATCH, 3))
